# baseline (speedup 1.0000x reference)
import jax
import jax.numpy as jnp
from jax import lax
from jax.experimental import pallas as pl
from jax.experimental.pallas import tpu as pltpu
from jax.experimental.pallas import tpu_sc as plsc

_NUM_CLASSES = 1000
_FEAT = 2048
_BATCH = 16384
_SIGMA = 0.5

_NC = 2
_NS = 16
_NW = _NC * _NS
_L = 16
_COLS = _FEAT // _NW
_CW = _COLS // _L
_CHUNK = 128
_NCHUNK = _BATCH // _CHUNK
_CPAD = 1024
_FIN = 64
_NFIN = 16


def _sc_body(feats_hbm, lab_hbm, mem_hbm,
             out_hbm,
             table, cnt2, fbuf, lbuf, alpha_v, beta_v, obuf,
             semf0, semf1, seml0, seml1):
  c = lax.axis_index("c")
  s = lax.axis_index("s")
  w = c * _NS + s
  col0 = w * _COLS

  iota = lax.iota(jnp.int32, _L)
  zvec = jnp.zeros((_L,), jnp.float32)
  ones = zvec + 1.0
  koffs = [iota + k * _L for k in range(_CW)]

  sems_f = (semf0, semf1)
  sems_l = (seml0, seml1)


  def _zt(i, _):
    table[pl.ds(i * _L, _L)] = zvec
    return 0
  lax.fori_loop(0, _NUM_CLASSES * _COLS // _L, _zt, 0)

  def _zc(i, _):
    cnt2[pl.ds(i * _L, _L)] = zvec
    return 0
  lax.fori_loop(0, _CPAD, _zc, 0)

  def _start(g, b):
    rows = pl.ds(g * _CHUNK, _CHUNK)
    pltpu.async_copy(feats_hbm.at[rows, pl.ds(col0, _COLS)], fbuf.at[b],
                     sems_f[b])
    pltpu.async_copy(lab_hbm.at[rows], lbuf.at[b], sems_l[b])

  _start(0, 0)

  def _chunk_loop(i, _):
    for b in range(2):
      g = 2 * i + b

      @pl.when(g + 1 < _NCHUNK)
      def _():
        _start(g + 1, 1 - b)

      pltpu.make_async_copy(
          feats_hbm.at[pl.ds(0, _CHUNK), pl.ds(0, _COLS)], fbuf.at[b],
          sems_f[b]).wait()
      pltpu.make_async_copy(lab_hbm.at[pl.ds(0, _CHUNK)], lbuf.at[b],
                            sems_l[b]).wait()

      def _group(gi, _):
        r0 = gi * _L
        lblv = lbuf[b, pl.ds(r0, _L)]
        plsc.addupdate_scatter(cnt2, [lblv * _L + iota], ones)
        basev = lblv * _COLS

        def _ld(r2):
          vals = [fbuf[b, r0 + r2, pl.ds(k * _L, _L)] for k in range(_CW)]
          bb = jnp.take_along_axis(
              basev, jnp.full((_L,), r2, jnp.int32), axis=0,
              mode="promise_in_bounds")
          return vals, [bb + koffs[k] for k in range(_CW)]

        vals, idxs = _ld(0)
        for r2 in range(1, _L + 1):
          nxt = _ld(r2) if r2 < _L else None
          for k in range(_CW):
            plsc.addupdate_scatter(table, [idxs[k]], vals[k])
          if nxt is not None:
            vals, idxs = nxt
        return 0
      lax.fori_loop(0, _CHUNK // _L, _group, 0)
    return 0

  lax.fori_loop(0, _NCHUNK // 2, _chunk_loop, 0)

  def _lred(gc, _):
    acc = zvec
    rowbase = (gc * _L + iota) * _L
    for j in range(_L):
      acc = acc + plsc.load_gather(cnt2, [rowbase + j])
    present = acc > 0.0
    at = pl.ds(gc * _L, _L)
    alpha_v[at] = jnp.where(present, 1.0 - _SIGMA, 1.0)
    beta_v[at] = jnp.where(present, _SIGMA / jnp.maximum(acc, 1.0), 0.0)
    return 0
  lax.fori_loop(0, _CPAD // _L, _lred, 0)

  cols = pl.ds(col0, _COLS)
  mbufs = (fbuf.at[0, pl.ds(0, _FIN)], fbuf.at[1, pl.ds(0, _FIN)])

  def _mstart(q, b):
    cls0 = jnp.minimum(q * _FIN, _NUM_CLASSES - _FIN)
    pltpu.async_copy(mem_hbm.at[pl.ds(cls0, _FIN), cols], mbufs[b],
                     sems_f[b])

  _mstart(0, 0)

  def _fin(i, _):
    for b in range(2):
      q = 2 * i + b
      cls0 = jnp.minimum(q * _FIN, _NUM_CLASSES - _FIN)

      @pl.when(q + 1 < _NFIN)
      def _():
        _mstart(q + 1, 1 - b)

      pltpu.make_async_copy(mem_hbm.at[pl.ds(0, _FIN), cols], mbufs[b],
                            sems_f[b]).wait()

      def _grp(g, _):
        c0 = cls0 + g * _L
        av = alpha_v[pl.ds(c0, _L)]
        bv = beta_v[pl.ds(c0, _L)]
        for r2 in range(_L):
          rsel = jnp.full((_L,), r2, jnp.int32)
          alpha = jnp.take_along_axis(av, rsel, axis=0,
                                      mode="promise_in_bounds")
          beta = jnp.take_along_axis(bv, rsel, axis=0,
                                     mode="promise_in_bounds")
          r = g * _L + r2
          tb = (c0 + r2) * _COLS
          for k in range(_CW):
            sv = table[pl.ds(tb + k * _L, _L)]
            mv = mbufs[b][r, pl.ds(k * _L, _L)]
            obuf[r, pl.ds(k * _L, _L)] = alpha * mv + beta * sv
        return 0
      lax.fori_loop(0, _FIN // _L, _grp, 0)

      pltpu.sync_copy(obuf, out_hbm.at[pl.ds(cls0, _FIN), cols])
    return 0

  lax.fori_loop(0, _NFIN // 2, _fin, 0)


@jax.jit
def kernel(rgb_feats, ir_feats, rgb_labels, ir_labels, vis_memory, ir_memory):
  mesh = plsc.VectorSubcoreMesh(core_axis_name="c", subcore_axis_name="s")
  one = pl.kernel(
      _sc_body,
      out_type=jax.ShapeDtypeStruct((_NUM_CLASSES, _FEAT), jnp.float32),
      mesh=mesh,
      compiler_params=pltpu.CompilerParams(use_tc_tiling_on_sc=False,
                                           needs_layout_passes=False),
      scratch_types=[
          pltpu.VMEM((_NUM_CLASSES * _COLS,), jnp.float32),
          pltpu.VMEM((_CPAD * _L,), jnp.float32),
          pltpu.VMEM((2, _CHUNK, _COLS), jnp.float32),
          pltpu.VMEM((2, _CHUNK), jnp.int32),
          pltpu.VMEM((_CPAD,), jnp.float32),
          pltpu.VMEM((_CPAD,), jnp.float32),
          pltpu.VMEM((_FIN, _COLS), jnp.float32),
          pltpu.SemaphoreType.DMA,
          pltpu.SemaphoreType.DMA,
          pltpu.SemaphoreType.DMA,
          pltpu.SemaphoreType.DMA,
      ],
  )
  new_vis = one(rgb_feats, rgb_labels, vis_memory)
  new_ir = one(ir_feats, ir_labels, ir_memory)
  return jnp.stack([new_vis, new_ir], axis=0)


# speedup vs baseline: 1.0904x; 1.0904x over previous
import jax
import jax.numpy as jnp
from jax import lax
from jax.experimental import pallas as pl
from jax.experimental.pallas import tpu as pltpu
from jax.experimental.pallas import tpu_sc as plsc

_NUM_CLASSES = 1000
_FEAT = 2048
_BATCH = 16384
_SIGMA = 0.5

_NC = 2
_NS = 16
_NW = _NC * _NS
_L = 16
_COLS = _FEAT // _NW
_CW = _COLS // _L
_CHUNK = 128
_NCHUNK = _BATCH // _CHUNK
_CPAD = 1024
_FIN = 64
_NFIN = 16


def _sc_body(rgb_hbm, ir_hbm, rgb_lab_hbm, ir_lab_hbm, vis_hbm, ir_mem_hbm,
             out_hbm,
             table, cnt2, fbuf, lbuf, alpha_v, beta_v, obuf,
             semf0, semf1, seml0, seml1):
  c = lax.axis_index("c")
  s = lax.axis_index("s")
  w = c * _NS + s
  col0 = w * _COLS

  iota = lax.iota(jnp.int32, _L)
  zvec = jnp.zeros((_L,), jnp.float32)
  ones = zvec + 1.0
  koffs = [iota + k * _L for k in range(_CW)]

  sems_f = (semf0, semf1)
  sems_l = (seml0, seml1)

  for m in range(2):
    feats_hbm = rgb_hbm if m == 0 else ir_hbm
    lab_hbm = rgb_lab_hbm if m == 0 else ir_lab_hbm
    mem_hbm = vis_hbm if m == 0 else ir_mem_hbm

    def _zt(i, _):
      table[pl.ds(i * _L, _L)] = zvec
      return 0
    lax.fori_loop(0, _NUM_CLASSES * _COLS // _L, _zt, 0, unroll=8)

    def _zc(i, _):
      cnt2[pl.ds(i * _L, _L)] = zvec
      return 0
    lax.fori_loop(0, _CPAD, _zc, 0, unroll=8)

    def _start(g, b):
      rows = pl.ds(g * _CHUNK, _CHUNK)
      pltpu.async_copy(feats_hbm.at[rows, pl.ds(col0, _COLS)], fbuf.at[b],
                       sems_f[b])
      pltpu.async_copy(lab_hbm.at[rows], lbuf.at[b], sems_l[b])

    _start(0, 0)

    def _chunk_loop(i, _):
      for b in range(2):
        g = 2 * i + b

        @pl.when(g + 1 < _NCHUNK)
        def _():
          _start(g + 1, 1 - b)

        pltpu.make_async_copy(
            feats_hbm.at[pl.ds(0, _CHUNK), pl.ds(0, _COLS)], fbuf.at[b],
            sems_f[b]).wait()
        pltpu.make_async_copy(lab_hbm.at[pl.ds(0, _CHUNK)], lbuf.at[b],
                              sems_l[b]).wait()

        def _group(gi, _):
          r0 = gi * _L
          lblv = lbuf[b, pl.ds(r0, _L)]
          plsc.addupdate_scatter(cnt2, [lblv * _L + iota], ones)
          basev = lblv * _COLS

          def _ld(r2):
            vals = [fbuf[b, r0 + r2, pl.ds(k * _L, _L)] for k in range(_CW)]
            bb = jnp.take_along_axis(
                basev, jnp.full((_L,), r2, jnp.int32), axis=0,
                mode="promise_in_bounds")
            return vals, [bb + koffs[k] for k in range(_CW)]

          vals, idxs = _ld(0)
          for r2 in range(1, _L + 1):
            nxt = _ld(r2) if r2 < _L else None
            for k in range(_CW):
              plsc.addupdate_scatter(table, [idxs[k]], vals[k])
            if nxt is not None:
              vals, idxs = nxt
          return 0
        lax.fori_loop(0, _CHUNK // _L, _group, 0, unroll=2)
      return 0

    lax.fori_loop(0, _NCHUNK // 2, _chunk_loop, 0)

    def _lred(gc, _):
      acc = zvec
      rowbase = (gc * _L + iota) * _L
      for j in range(_L):
        acc = acc + plsc.load_gather(cnt2, [rowbase + j])
      present = acc > 0.0
      at = pl.ds(gc * _L, _L)
      alpha_v[at] = jnp.where(present, 1.0 - _SIGMA, 1.0)
      beta_v[at] = jnp.where(present, _SIGMA / jnp.maximum(acc, 1.0), 0.0)
      return 0
    lax.fori_loop(0, _CPAD // _L, _lred, 0)

    cols = pl.ds(col0, _COLS)
    mbufs = (fbuf.at[0, pl.ds(0, _FIN)], fbuf.at[1, pl.ds(0, _FIN)])

    def _mstart(q, b):
      cls0 = jnp.minimum(q * _FIN, _NUM_CLASSES - _FIN)
      pltpu.async_copy(mem_hbm.at[pl.ds(cls0, _FIN), cols], mbufs[b],
                       sems_f[b])

    _mstart(0, 0)

    def _fin(i, _):
      for b in range(2):
        q = 2 * i + b
        cls0 = jnp.minimum(q * _FIN, _NUM_CLASSES - _FIN)

        @pl.when(q + 1 < _NFIN)
        def _():
          _mstart(q + 1, 1 - b)

        pltpu.make_async_copy(mem_hbm.at[pl.ds(0, _FIN), cols], mbufs[b],
                              sems_f[b]).wait()

        def _grp(g, _):
          c0 = cls0 + g * _L
          av = alpha_v[pl.ds(c0, _L)]
          bv = beta_v[pl.ds(c0, _L)]
          for r2 in range(_L):
            rsel = jnp.full((_L,), r2, jnp.int32)
            alpha = jnp.take_along_axis(av, rsel, axis=0,
                                        mode="promise_in_bounds")
            beta = jnp.take_along_axis(bv, rsel, axis=0,
                                       mode="promise_in_bounds")
            r = g * _L + r2
            tb = (c0 + r2) * _COLS
            for k in range(_CW):
              sv = table[pl.ds(tb + k * _L, _L)]
              mv = mbufs[b][r, pl.ds(k * _L, _L)]
              obuf[r, pl.ds(k * _L, _L)] = alpha * mv + beta * sv
          return 0
        lax.fori_loop(0, _FIN // _L, _grp, 0)

        pltpu.sync_copy(obuf, out_hbm.at[m, pl.ds(cls0, _FIN), cols])
      return 0

    lax.fori_loop(0, _NFIN // 2, _fin, 0)


@jax.jit
def kernel(rgb_feats, ir_feats, rgb_labels, ir_labels, vis_memory, ir_memory):
  mesh = plsc.VectorSubcoreMesh(core_axis_name="c", subcore_axis_name="s")
  return pl.kernel(
      _sc_body,
      out_type=jax.ShapeDtypeStruct((2, _NUM_CLASSES, _FEAT), jnp.float32),
      mesh=mesh,
      compiler_params=pltpu.CompilerParams(use_tc_tiling_on_sc=False,
                                           needs_layout_passes=False),
      scratch_types=[
          pltpu.VMEM((_NUM_CLASSES * _COLS,), jnp.float32),
          pltpu.VMEM((_CPAD * _L,), jnp.float32),
          pltpu.VMEM((2, _CHUNK, _COLS), jnp.float32),
          pltpu.VMEM((2, _CHUNK), jnp.int32),
          pltpu.VMEM((_CPAD,), jnp.float32),
          pltpu.VMEM((_CPAD,), jnp.float32),
          pltpu.VMEM((_FIN, _COLS), jnp.float32),
          pltpu.SemaphoreType.DMA,
          pltpu.SemaphoreType.DMA,
          pltpu.SemaphoreType.DMA,
          pltpu.SemaphoreType.DMA,
      ],
  )(rgb_feats, ir_feats, rgb_labels, ir_labels, vis_memory, ir_memory)


# speedup vs baseline: 1.1590x; 1.0630x over previous
import jax
import jax.numpy as jnp
from jax import lax
from jax.experimental import pallas as pl
from jax.experimental.pallas import tpu as pltpu
from jax.experimental.pallas import tpu_sc as plsc

_NUM_CLASSES = 1000
_FEAT = 2048
_BATCH = 16384
_SIGMA = 0.5

_NC = 2
_NS = 16
_NW = _NC * _NS
_L = 16
_COLS = _FEAT // _NW
_CW = _COLS // _L
_CHUNK = 256
_NCHUNK = _BATCH // _CHUNK
_CPAD = 1024
_FIN = 64
_NFIN = 16


def _sc_body(rgb_hbm, ir_hbm, rgb_lab_hbm, ir_lab_hbm, vis_hbm, ir_mem_hbm,
             out_hbm,
             table, cnt2, fbuf, lbuf, alpha_v, beta_v, obuf,
             semf0, semf1, seml0, seml1):
  c = lax.axis_index("c")
  s = lax.axis_index("s")
  w = c * _NS + s
  col0 = w * _COLS

  iota = lax.iota(jnp.int32, _L)
  zvec = jnp.zeros((_L,), jnp.float32)
  ones = zvec + 1.0
  koffs = [iota + k * _L for k in range(_CW)]

  sems_f = (semf0, semf1)
  sems_l = (seml0, seml1)

  for m in range(2):
    feats_hbm = rgb_hbm if m == 0 else ir_hbm
    lab_hbm = rgb_lab_hbm if m == 0 else ir_lab_hbm
    mem_hbm = vis_hbm if m == 0 else ir_mem_hbm

    def _zt(i, _):
      table[pl.ds(i * _L, _L)] = zvec
      return 0
    lax.fori_loop(0, _NUM_CLASSES * _COLS // _L, _zt, 0, unroll=8)

    def _zc(i, _):
      cnt2[pl.ds(i * _L, _L)] = zvec
      return 0
    lax.fori_loop(0, _CPAD, _zc, 0, unroll=8)

    def _start(g, b):
      rows = pl.ds(g * _CHUNK, _CHUNK)
      pltpu.async_copy(feats_hbm.at[rows, pl.ds(col0, _COLS)], fbuf.at[b],
                       sems_f[b])
      pltpu.async_copy(lab_hbm.at[rows], lbuf.at[b], sems_l[b])

    _start(0, 0)

    def _chunk_loop(i, _):
      for b in range(2):
        g = 2 * i + b

        @pl.when(g + 1 < _NCHUNK)
        def _():
          _start(g + 1, 1 - b)

        pltpu.make_async_copy(
            feats_hbm.at[pl.ds(0, _CHUNK), pl.ds(0, _COLS)], fbuf.at[b],
            sems_f[b]).wait()
        pltpu.make_async_copy(lab_hbm.at[pl.ds(0, _CHUNK)], lbuf.at[b],
                              sems_l[b]).wait()

        def _group(gi, _):
          r0 = gi * _L
          lblv = lbuf[b, pl.ds(r0, _L)]
          plsc.addupdate_scatter(cnt2, [lblv * _L + iota], ones)
          basev = lblv * _COLS

          def _ld(r2):
            vals = [fbuf[b, r0 + r2, pl.ds(k * _L, _L)] for k in range(_CW)]
            bb = jnp.take_along_axis(
                basev, jnp.full((_L,), r2, jnp.int32), axis=0,
                mode="promise_in_bounds")
            return vals, [bb + koffs[k] for k in range(_CW)]

          vals, idxs = _ld(0)
          for r2 in range(1, _L + 1):
            nxt = _ld(r2) if r2 < _L else None
            for k in range(_CW):
              plsc.addupdate_scatter(table, [idxs[k]], vals[k])
            if nxt is not None:
              vals, idxs = nxt
          return 0
        lax.fori_loop(0, _CHUNK // _L, _group, 0, unroll=2)
      return 0

    lax.fori_loop(0, _NCHUNK // 2, _chunk_loop, 0)

    def _lred(gc, _):
      acc = zvec
      rowbase = (gc * _L + iota) * _L
      for j in range(_L):
        acc = acc + plsc.load_gather(cnt2, [rowbase + j])
      present = acc > 0.0
      at = pl.ds(gc * _L, _L)
      alpha_v[at] = jnp.where(present, 1.0 - _SIGMA, 1.0)
      beta_v[at] = jnp.where(present, _SIGMA / jnp.maximum(acc, 1.0), 0.0)
      return 0
    lax.fori_loop(0, _CPAD // _L, _lred, 0)

    cols = pl.ds(col0, _COLS)
    mbufs = (fbuf.at[0, pl.ds(0, _FIN)], fbuf.at[1, pl.ds(0, _FIN)])

    def _mstart(q, b):
      cls0 = jnp.minimum(q * _FIN, _NUM_CLASSES - _FIN)
      pltpu.async_copy(mem_hbm.at[pl.ds(cls0, _FIN), cols], mbufs[b],
                       sems_f[b])

    _mstart(0, 0)

    def _fin(i, _):
      for b in range(2):
        q = 2 * i + b
        cls0 = jnp.minimum(q * _FIN, _NUM_CLASSES - _FIN)

        @pl.when(q + 1 < _NFIN)
        def _():
          _mstart(q + 1, 1 - b)

        pltpu.make_async_copy(mem_hbm.at[pl.ds(0, _FIN), cols], mbufs[b],
                              sems_f[b]).wait()

        def _grp(g, _):
          c0 = cls0 + g * _L
          av = alpha_v[pl.ds(c0, _L)]
          bv = beta_v[pl.ds(c0, _L)]
          for r2 in range(_L):
            rsel = jnp.full((_L,), r2, jnp.int32)
            alpha = jnp.take_along_axis(av, rsel, axis=0,
                                        mode="promise_in_bounds")
            beta = jnp.take_along_axis(bv, rsel, axis=0,
                                       mode="promise_in_bounds")
            r = g * _L + r2
            tb = (c0 + r2) * _COLS
            for k in range(_CW):
              sv = table[pl.ds(tb + k * _L, _L)]
              mv = mbufs[b][r, pl.ds(k * _L, _L)]
              obuf[r, pl.ds(k * _L, _L)] = alpha * mv + beta * sv
          return 0
        lax.fori_loop(0, _FIN // _L, _grp, 0, unroll=2)

        pltpu.sync_copy(obuf, out_hbm.at[m, pl.ds(cls0, _FIN), cols])
      return 0

    lax.fori_loop(0, _NFIN // 2, _fin, 0)


@jax.jit
def kernel(rgb_feats, ir_feats, rgb_labels, ir_labels, vis_memory, ir_memory):
  mesh = plsc.VectorSubcoreMesh(core_axis_name="c", subcore_axis_name="s")
  return pl.kernel(
      _sc_body,
      out_type=jax.ShapeDtypeStruct((2, _NUM_CLASSES, _FEAT), jnp.float32),
      mesh=mesh,
      compiler_params=pltpu.CompilerParams(use_tc_tiling_on_sc=False,
                                           needs_layout_passes=False),
      scratch_types=[
          pltpu.VMEM((_NUM_CLASSES * _COLS,), jnp.float32),
          pltpu.VMEM((_CPAD * _L,), jnp.float32),
          pltpu.VMEM((2, _CHUNK, _COLS), jnp.float32),
          pltpu.VMEM((2, _CHUNK), jnp.int32),
          pltpu.VMEM((_CPAD,), jnp.float32),
          pltpu.VMEM((_CPAD,), jnp.float32),
          pltpu.VMEM((_FIN, _COLS), jnp.float32),
          pltpu.SemaphoreType.DMA,
          pltpu.SemaphoreType.DMA,
          pltpu.SemaphoreType.DMA,
          pltpu.SemaphoreType.DMA,
      ],
  )(rgb_feats, ir_feats, rgb_labels, ir_labels, vis_memory, ir_memory)
